# initial kernel scaffold (unmeasured)
import jax
import jax.numpy as jnp
from jax import lax
from jax.experimental import pallas as pl
from jax.experimental.pallas import tpu as pltpu

B, H, D, BS = 32, 16, 128, 32
NEG = -1e30


def _compute_body(q_ref, k_ref, v_ref, bt_ref, lens_ref, o_ref, m_ref, l_ref):
    c = pl.program_id(0)
    n_pages = k_ref.shape[0]
    t = n_pages * BS

    @pl.when(c == 0)
    def _():
        o_ref[...] = jnp.zeros_like(o_ref)
        m_ref[...] = jnp.full_like(m_ref, NEG)
        l_ref[...] = jnp.zeros_like(l_ref)

    z = lax.axis_index("z")
    base = z * 256 + c * n_pages

    bt = bt_ref[...]
    jidx = lax.broadcasted_iota(jnp.int32, bt.shape, 1)
    valid = jidx < lens_ref[...]
    pidx = lax.broadcasted_iota(
        jnp.int32, bt.shape + (n_pages,), 2) + base
    eq = jnp.logical_and(bt[:, :, None] == pidx, valid[:, :, None])
    counts = jnp.sum(eq.astype(jnp.float32), axis=1)
    w = jnp.broadcast_to(counts[:, :, None], (B, n_pages, BS)).reshape(B, t)

    q = (q_ref[...] * (D ** -0.5)).astype(jnp.bfloat16)
    k = k_ref[...].reshape(t, H, D).astype(jnp.bfloat16)
    s = lax.dot_general(
        q, k,
        dimension_numbers=(((2,), (2,)), ((1,), (1,))),
        preferred_element_type=jnp.float32,
    )

    wb = w[None, :, :]
    s_masked = jnp.where(wb > 0.0, s, NEG)
    m_c = jnp.max(s_masked, axis=2)
    m_old = m_ref[...]
    m_new = jnp.maximum(m_old, m_c)
    p = wb * jnp.exp(jnp.minimum(s - m_new[:, :, None], 0.0))
    scale = jnp.exp(m_old - m_new)

    l_ref[...] = l_ref[...] * scale + jnp.sum(p, axis=2)
    v = v_ref[...].reshape(t, H, D).astype(jnp.bfloat16)
    o_c = lax.dot_general(
        p.astype(jnp.bfloat16), v,
        dimension_numbers=(((2,), (0,)), ((0,), (1,))),
        preferred_element_type=jnp.float32,
    )
    o_ref[...] = o_ref[...] * scale[:, :, None] + o_c
    m_ref[...] = m_new


def _exchange_body(o_ref, s_ref, out_ref, comm_o, comm_s, send_sems, recv_sems):
    x = lax.axis_index("x")
    y = lax.axis_index("y")
    z = lax.axis_index("z")
    nbr = (x, y, 1 - z)

    barrier_sem = pltpu.get_barrier_semaphore()
    pl.semaphore_signal(
        barrier_sem, inc=1, device_id=nbr,
        device_id_type=pl.DeviceIdType.MESH,
    )
    pl.semaphore_wait(barrier_sem, 1)

    rdma_o = pltpu.make_async_remote_copy(
        src_ref=o_ref, dst_ref=comm_o,
        send_sem=send_sems.at[0], recv_sem=recv_sems.at[0],
        device_id=nbr, device_id_type=pl.DeviceIdType.MESH,
    )
    rdma_s = pltpu.make_async_remote_copy(
        src_ref=s_ref, dst_ref=comm_s,
        send_sem=send_sems.at[1], recv_sem=recv_sems.at[1],
        device_id=nbr, device_id_type=pl.DeviceIdType.MESH,
    )
    rdma_o.start()
    rdma_s.start()
    rdma_o.wait()
    rdma_s.wait()

    m0 = s_ref[0:H, :]
    l0 = s_ref[H:2 * H, :]
    m1 = comm_s[0:H, :]
    l1 = comm_s[H:2 * H, :]
    mg = jnp.maximum(m0, m1)
    a0 = jnp.exp(m0 - mg)
    a1 = jnp.exp(m1 - mg)
    lg = l0 * a0 + l1 * a1
    o = (o_ref[...] * a0[:, :, None] + comm_o[...] * a1[:, :, None]) \
        / lg[:, :, None]
    out_ref[...] = jnp.transpose(o, (1, 0, 2)).reshape(B, 1, H, D)


def kernel(Q, K, V, bt, lens):
    n_local_pages = K.shape[0]
    chunk_pages = 32
    n_chunks = n_local_pages // chunk_pages

    qb = Q.reshape(B, H, D)
    lens2 = lens.reshape(B, 1)

    o_un, m, l = pl.pallas_call(
        _compute_body,
        grid=(n_chunks,),
        in_specs=[
            pl.BlockSpec((B, H, D), lambda c: (0, 0, 0)),
            pl.BlockSpec((chunk_pages, BS, H, D), lambda c: (c, 0, 0, 0)),
            pl.BlockSpec((chunk_pages, BS, H, D), lambda c: (c, 0, 0, 0)),
            pl.BlockSpec(bt.shape, lambda c: (0, 0)),
            pl.BlockSpec((B, 1), lambda c: (0, 0)),
        ],
        out_specs=[
            pl.BlockSpec((H, B, D), lambda c: (0, 0, 0)),
            pl.BlockSpec((H, B), lambda c: (0, 0)),
            pl.BlockSpec((H, B), lambda c: (0, 0)),
        ],
        out_shape=[
            jax.ShapeDtypeStruct((H, B, D), jnp.float32),
            jax.ShapeDtypeStruct((H, B), jnp.float32),
            jax.ShapeDtypeStruct((H, B), jnp.float32),
        ],
    )(qb, K, V, bt, lens2)

    stats = jnp.concatenate([m, l], axis=0)

    return pl.pallas_call(
        _exchange_body,
        out_shape=jax.ShapeDtypeStruct((B, 1, H, D), jnp.float32),
        in_specs=[
            pl.BlockSpec(memory_space=pltpu.VMEM),
            pl.BlockSpec(memory_space=pltpu.VMEM),
        ],
        out_specs=pl.BlockSpec(memory_space=pltpu.VMEM),
        scratch_shapes=[
            pltpu.VMEM((H, B, D), jnp.float32),
            pltpu.VMEM((2 * H, B), jnp.float32),
            pltpu.SemaphoreType.DMA((2,)),
            pltpu.SemaphoreType.DMA((2,)),
        ],
        compiler_params=pltpu.CompilerParams(collective_id=0),
    )(o_un, stats)


# baseline (device time: 193748 ns/iter reference)
import jax
import jax.numpy as jnp
from jax import lax
from jax.experimental import pallas as pl
from jax.experimental.pallas import tpu as pltpu

B, H, D, BS = 32, 16, 128, 32
NEG = -1e30


def _compute_body(q_ref, k_ref, v_ref, bt_ref, lens_ref, o_ref, m_ref, l_ref):
    c = pl.program_id(0)
    n_pages = k_ref.shape[0]
    t = n_pages * BS

    @pl.when(c == 0)
    def _():
        o_ref[...] = jnp.zeros_like(o_ref)
        m_ref[...] = jnp.full_like(m_ref, NEG)
        l_ref[...] = jnp.zeros_like(l_ref)

    z = lax.axis_index("z")
    base = z * 256 + c * n_pages

    bt = bt_ref[...]
    nb = bt.shape[1]
    jidx = lax.broadcasted_iota(jnp.int32, bt.shape, 1)
    validf = (jidx < lens_ref[...]).astype(jnp.float32)
    bt3 = lax.broadcast_in_dim(bt, (B, n_pages, nb), (0, 2))
    valid3 = lax.broadcast_in_dim(validf, (B, n_pages, nb), (0, 2))
    pidx3 = lax.broadcasted_iota(jnp.int32, (B, n_pages, nb), 1) + base
    eqf = (bt3 == pidx3).astype(jnp.float32) * valid3
    counts = jnp.sum(eqf, axis=2)
    w = jnp.broadcast_to(counts[:, :, None], (B, n_pages, BS)).reshape(B, t)

    q = (q_ref[...] * (D ** -0.5)).astype(jnp.bfloat16)
    k = k_ref[...].reshape(t, H, D).astype(jnp.bfloat16)
    s = lax.dot_general(
        q, k,
        dimension_numbers=(((2,), (2,)), ((1,), (1,))),
        preferred_element_type=jnp.float32,
    )

    wb = w[None, :, :]
    s_masked = jnp.where(wb > 0.0, s, NEG)
    m_c = jnp.max(s_masked, axis=2)
    m_old = m_ref[...]
    m_new = jnp.maximum(m_old, m_c)
    p = wb * jnp.exp(jnp.minimum(s - m_new[:, :, None], 0.0))
    scale = jnp.exp(m_old - m_new)

    l_ref[...] = l_ref[...] * scale + jnp.sum(p, axis=2)
    v = v_ref[...].reshape(t, H, D).astype(jnp.bfloat16)
    o_c = lax.dot_general(
        p.astype(jnp.bfloat16), v,
        dimension_numbers=(((2,), (0,)), ((0,), (1,))),
        preferred_element_type=jnp.float32,
    )
    o_ref[...] = o_ref[...] * scale[:, :, None] + o_c
    m_ref[...] = m_new


def _exchange_body(o_ref, s_ref, out_ref, comm_o, comm_s, send_sems, recv_sems):
    x = lax.axis_index("x")
    y = lax.axis_index("y")
    z = lax.axis_index("z")
    nbr = (x, y, 1 - z)

    barrier_sem = pltpu.get_barrier_semaphore()
    pl.semaphore_signal(
        barrier_sem, inc=1, device_id=nbr,
        device_id_type=pl.DeviceIdType.MESH,
    )
    pl.semaphore_wait(barrier_sem, 1)

    rdma_o = pltpu.make_async_remote_copy(
        src_ref=o_ref, dst_ref=comm_o,
        send_sem=send_sems.at[0], recv_sem=recv_sems.at[0],
        device_id=nbr, device_id_type=pl.DeviceIdType.MESH,
    )
    rdma_s = pltpu.make_async_remote_copy(
        src_ref=s_ref, dst_ref=comm_s,
        send_sem=send_sems.at[1], recv_sem=recv_sems.at[1],
        device_id=nbr, device_id_type=pl.DeviceIdType.MESH,
    )
    rdma_o.start()
    rdma_s.start()
    rdma_o.wait()
    rdma_s.wait()

    m0 = s_ref[0:H, :]
    l0 = s_ref[H:2 * H, :]
    m1 = comm_s[0:H, :]
    l1 = comm_s[H:2 * H, :]
    mg = jnp.maximum(m0, m1)
    a0 = jnp.exp(m0 - mg)
    a1 = jnp.exp(m1 - mg)
    lg = l0 * a0 + l1 * a1
    o = (o_ref[...] * a0[:, :, None] + comm_o[...] * a1[:, :, None]) \
        / lg[:, :, None]
    out_ref[...] = jnp.transpose(o, (1, 0, 2)).reshape(B, 1, H, D)


def kernel(Q, K, V, bt, lens):
    n_local_pages = K.shape[0]
    chunk_pages = 8
    n_chunks = n_local_pages // chunk_pages

    qb = Q.reshape(B, H, D)
    lens2 = lens.reshape(B, 1)

    o_un, m, l = pl.pallas_call(
        _compute_body,
        grid=(n_chunks,),
        in_specs=[
            pl.BlockSpec((B, H, D), lambda c: (0, 0, 0)),
            pl.BlockSpec((chunk_pages, BS, H, D), lambda c: (c, 0, 0, 0)),
            pl.BlockSpec((chunk_pages, BS, H, D), lambda c: (c, 0, 0, 0)),
            pl.BlockSpec(bt.shape, lambda c: (0, 0)),
            pl.BlockSpec((B, 1), lambda c: (0, 0)),
        ],
        out_specs=[
            pl.BlockSpec((H, B, D), lambda c: (0, 0, 0)),
            pl.BlockSpec((H, B), lambda c: (0, 0)),
            pl.BlockSpec((H, B), lambda c: (0, 0)),
        ],
        out_shape=[
            jax.ShapeDtypeStruct((H, B, D), jnp.float32),
            jax.ShapeDtypeStruct((H, B), jnp.float32),
            jax.ShapeDtypeStruct((H, B), jnp.float32),
        ],
    )(qb, K, V, bt, lens2)

    stats = jnp.concatenate([m, l], axis=0)

    return pl.pallas_call(
        _exchange_body,
        out_shape=jax.ShapeDtypeStruct((B, 1, H, D), jnp.float32),
        in_specs=[
            pl.BlockSpec(memory_space=pltpu.VMEM),
            pl.BlockSpec(memory_space=pltpu.VMEM),
        ],
        out_specs=pl.BlockSpec(memory_space=pltpu.VMEM),
        scratch_shapes=[
            pltpu.VMEM((H, B, D), jnp.float32),
            pltpu.VMEM((2 * H, B), jnp.float32),
            pltpu.SemaphoreType.DMA((2,)),
            pltpu.SemaphoreType.DMA((2,)),
        ],
        compiler_params=pltpu.CompilerParams(collective_id=0),
    )(o_un, stats)
